# baseline (speedup 1.0000x reference)
import functools
import math

import jax
import jax.numpy as jnp
from jax import lax
from jax.experimental import pallas as pl
from jax.experimental.pallas import tpu as pltpu
from jax.experimental.pallas import tpu_sc as plsc

D_MODEL = 64
PAIR_W = 2 * D_MODEL
SCALE = math.sqrt(D_MODEL)
LANES = 16
NBUF = 2
RING_ROWS = 2
RP_ROWS = 256


@functools.lru_cache(maxsize=None)
def _make_repack(V):
    info = plsc.get_sparse_core_info()
    nw = info.num_cores * info.num_subcores
    mesh = plsc.VectorSubcoreMesh(core_axis_name="c", subcore_axis_name="s")
    stride_w = (V // nw) // 8 * 8
    rows_w = V - stride_w * (nw - 1)
    n_steps = -(-rows_w // RP_ROWS)

    @functools.partial(
        pl.kernel,
        mesh=mesh,
        out_type=jax.ShapeDtypeStruct((V // 2, PAIR_W), jnp.float32),
        scratch_types=[
            *[pltpu.VMEM((RP_ROWS, D_MODEL), jnp.float32) for _ in range(2)],
            pltpu.VMEM((RP_ROWS // 2, PAIR_W), jnp.float32),
            *[pltpu.SemaphoreType.DMA for _ in range(2)],
        ],
        compiler_params=pltpu.CompilerParams(needs_layout_passes=False),
    )
    def repack(t_hbm, out_hbm, ib0, ib1, ob, sem0, sem1):
        ibs = (ib0, ib1)
        sems = (sem0, sem1)
        wid = lax.axis_index("s") * info.num_cores + lax.axis_index("c")
        start = wid * stride_w
        end = jnp.minimum(start + rows_w, V)

        def row0(s):
            return jnp.minimum(start + s * RP_ROWS, end - RP_ROWS)

        def read(s, b):
            pltpu.async_copy(
                t_hbm.at[pl.ds(pl.multiple_of(row0(s), 8), RP_ROWS)],
                ibs[b],
                sems[b],
            )

        def read_wait(s, b):
            pltpu.make_async_copy(
                t_hbm.at[pl.ds(pl.multiple_of(row0(s), 8), RP_ROWS)],
                ibs[b],
                sems[b],
            ).wait()

        for b in range(2):
            read(b, b)

        def do_step(s, b):
                read_wait(s, b)
                ib = ibs[b]

                def pack(q, c2):
                    r = q * 8
                    for rr in range(8):
                        for c in range(D_MODEL // LANES):
                            ob[
                                (r + rr) // 2,
                                pl.ds(((r + rr) % 2) * D_MODEL + c * LANES, LANES),
                            ] = ib[r + rr, pl.ds(c * LANES, LANES)]
                    return c2

                lax.fori_loop(0, RP_ROWS // 8, pack, 0)
                pltpu.sync_copy(
                    ob,
                    out_hbm.at[
                        pl.ds(
                            pl.multiple_of(row0(s) // 2, 8), RP_ROWS // 2
                        )
                    ],
                )

                @pl.when(s + 2 < n_steps)
                def _():
                    read(s + 2, b)

        def step(s2, carry):
            for b in range(2):
                do_step(s2 * 2 + b, b)
            return carry

        lax.fori_loop(0, n_steps // 2, step, 0)
        if n_steps % 2:
            do_step(n_steps - 1, (n_steps - 1) % 2)

    return repack


@functools.lru_cache(maxsize=None)
def _make_gather(R, T, V):
    info = plsc.get_sparse_core_info()
    nw = info.num_cores * info.num_subcores
    r_per_w = R // nw
    b_per_w = r_per_w * T
    half_t = T // 2
    mesh = plsc.VectorSubcoreMesh(core_axis_name="c", subcore_axis_name="s")

    @functools.partial(
        pl.kernel,
        mesh=mesh,
        out_type=jax.ShapeDtypeStruct((R * T // 2, PAIR_W), jnp.float32),
        scratch_types=[
            pltpu.VMEM((b_per_w,), jnp.int32),
            pltpu.VMEM((b_per_w // 32,), jnp.int32),
            pltpu.VMEM((RING_ROWS * half_t, PAIR_W), jnp.float32),
            *[pltpu.VMEM((T, PAIR_W), jnp.float32) for _ in range(NBUF)],
            *[pltpu.SemaphoreType.DMA for _ in range(NBUF)],
        ],
        compiler_params=pltpu.CompilerParams(needs_layout_passes=False),
    )
    def gather(xh_hbm, xp_hbm, t2_hbm, out_hbm, idx_v, par_v, obuf, *bufs_sems):
        bufs = bufs_sems[:NBUF]
        sems = bufs_sems[NBUF:]
        wid = lax.axis_index("s") * info.num_cores + lax.axis_index("c")
        base = wid * b_per_w

        pltpu.sync_copy(
            xh_hbm.at[pl.ds(pl.multiple_of(base, 128), b_per_w)], idx_v
        )
        pltpu.sync_copy(
            xp_hbm.at[pl.ds(pl.multiple_of(base // 32, 8), b_per_w // 32)],
            par_v,
        )

        def idx_list(j):
            return idx_v.at[pl.ds(j * T, T)]

        for b in range(NBUF):
            pltpu.async_copy(t2_hbm.at[idx_list(b)], bufs[b], sems[b])

        lane = lax.iota(jnp.int32, 16)

        def group_body(g, carry):
            for b in range(NBUF):
                j = g * NBUF + b
                buf = bufs[b]
                pltpu.make_async_copy(
                    t2_hbm.at[idx_list(j)], buf, sems[b]
                ).wait()

                orow0 = b * half_t

                def fix_tok(u2, c2):
                    for uu in range(2):
                        u = u2 * 2 + uu
                        for h in range(2):
                            t = u + h * half_t
                            pos = j * T + t
                            t16 = jnp.full((16,), t, jnp.int32)
                            word = plsc.load_gather(
                                par_v,
                                [jnp.full((16,), pos // 32, jnp.int32)],
                            )
                            off16 = ((word >> (pos % 32)) & 1) * D_MODEL + lane
                            for c in range(D_MODEL // LANES):
                                v = plsc.load_gather(
                                    buf, [t16, off16 + (c * LANES)]
                                )
                                obuf[
                                    orow0 + u,
                                    pl.ds(h * D_MODEL + c * LANES, LANES),
                                ] = v
                    return c2

                lax.fori_loop(0, half_t // 2, fix_tok, 0)

                if (b + 1) % RING_ROWS == 0:
                    orow_hbm = (base + (j - RING_ROWS + 1) * T) // 2
                    pltpu.sync_copy(
                        obuf,
                        out_hbm.at[
                            pl.ds(
                                pl.multiple_of(orow_hbm, 8),
                                RING_ROWS * half_t,
                            )
                        ],
                    )

                @pl.when(j + NBUF < r_per_w)
                def _():
                    pltpu.async_copy(
                        t2_hbm.at[idx_list(j + NBUF)], buf, sems[b]
                    )

            return carry

        lax.fori_loop(0, r_per_w // NBUF, group_body, 0)

    return gather


def _finish_body(g_ref, out_ref):
    nr = out_ref.shape[0]
    t = out_ref.shape[1]
    half = t // 2
    g = g_ref[...]
    a = g[:, 0:D_MODEL] * SCALE
    b = g[:, D_MODEL:PAIR_W] * SCALE
    for i in range(nr):
        out_ref[i, pl.ds(0, half), :] = a[i * half : (i + 1) * half, :]
        out_ref[i, pl.ds(half, half), :] = b[i * half : (i + 1) * half, :]


@functools.lru_cache(maxsize=None)
def _make_finish(R, T, tc_rows=16):
    grid = R // tc_rows
    return pl.pallas_call(
        _finish_body,
        grid=(grid,),
        in_specs=[
            pl.BlockSpec((tc_rows * T // 2, PAIR_W), lambda i: (i, 0)),
        ],
        out_specs=pl.BlockSpec((tc_rows, T, D_MODEL), lambda i: (i, 0, 0)),
        out_shape=jax.ShapeDtypeStruct((R, T, D_MODEL), jnp.float32),
        compiler_params=pltpu.CompilerParams(
            dimension_semantics=("arbitrary",)
        ),
    )


def kernel(x, table):
    R, T = x.shape
    B = R * T
    xf = x.reshape(B)
    xh = xf >> 1
    bits = (xf & 1).astype(jnp.uint32).reshape(B // 32, 32)
    xp = (
        (bits << jnp.arange(32, dtype=jnp.uint32))
        .sum(axis=1, dtype=jnp.uint32)
        .astype(jnp.int32)
    )
    t2 = _make_repack(table.shape[0])(table)
    g = _make_gather(R, T, table.shape[0])(xh, xp, t2)
    return _make_finish(R, T)(g)


# speedup vs baseline: 1.6807x; 1.6807x over previous
import functools
import math

import jax
import jax.numpy as jnp
from jax import lax
from jax.experimental import pallas as pl
from jax.experimental.pallas import tpu as pltpu
from jax.experimental.pallas import tpu_sc as plsc

D_MODEL = 64
SCALE = math.sqrt(D_MODEL)
LANES = 16
NBUF = 4
ROW_UNROLL = 4


@functools.lru_cache(maxsize=None)
def _make_emb(R, T):
    info = plsc.get_sparse_core_info()
    nw = info.num_cores * info.num_subcores
    r_per_w = R // nw
    mesh = plsc.VectorSubcoreMesh(core_axis_name="c", subcore_axis_name="s")

    @functools.partial(
        pl.kernel,
        mesh=mesh,
        out_type=jax.ShapeDtypeStruct((R, T, D_MODEL), jnp.float32),
        scratch_types=[
            pltpu.VMEM((r_per_w, T), jnp.int32),
            *[pltpu.VMEM((T, D_MODEL), jnp.float32) for _ in range(NBUF)],
            *[pltpu.SemaphoreType.DMA for _ in range(NBUF)],
        ],
        compiler_params=pltpu.CompilerParams(use_tc_tiling_on_sc=False),
    )
    def emb(x_hbm, table_hbm, out_hbm, idx_v, *bufs_sems):
        bufs = bufs_sems[:NBUF]
        sems = bufs_sems[NBUF:]
        wid = lax.axis_index("s") * info.num_cores + lax.axis_index("c")
        r_base = wid * r_per_w

        pltpu.sync_copy(x_hbm.at[pl.ds(r_base, r_per_w)], idx_v)

        for b in range(NBUF):
            pltpu.async_copy(table_hbm.at[idx_v.at[b]], bufs[b], sems[b])

        def group_body(g, carry):
            for b in range(NBUF):
                j = g * NBUF + b
                buf = bufs[b]
                pltpu.make_async_copy(
                    table_hbm.at[idx_v.at[j]], buf, sems[b]
                ).wait()

                def scale_rows(rq, c2):
                    r0 = rq * ROW_UNROLL
                    for rr in range(ROW_UNROLL):
                        for c in range(D_MODEL // LANES):
                            sl = pl.ds(c * LANES, LANES)
                            buf[r0 + rr, sl] = buf[r0 + rr, sl] * SCALE
                    return c2

                lax.fori_loop(0, T // ROW_UNROLL, scale_rows, 0)

                pltpu.sync_copy(buf, out_hbm.at[r_base + j])

                @pl.when(j + NBUF < r_per_w)
                def _():
                    pltpu.async_copy(table_hbm.at[idx_v.at[j + NBUF]], buf, sems[b])

            return carry

        lax.fori_loop(0, r_per_w // NBUF, group_body, 0)

    return emb


def kernel(x, table):
    return _make_emb(x.shape[0], x.shape[1])(x.astype(jnp.int32), table)
